# baseline (speedup 1.0000x reference)
import functools

import jax
import jax.numpy as jnp
from jax import lax
from jax.experimental import pallas as pl
from jax.experimental.pallas import tpu as pltpu
from jax.experimental.pallas import tpu_sc as plsc

VOCAB = 100000
EMBED_DIM = 128
BATCH = 4096
HIST_LEN = 50

_NC = 2
_NS = 16
_NW = _NC * _NS
_ROWS = BATCH * HIST_LEN
_RPW = _ROWS // _NW
_CHUNK = 128
_NCH = _RPW // _CHUNK
_NBUF = 5
_NROUNDS = _NCH // _NBUF

_mesh = plsc.VectorSubcoreMesh(core_axis_name="c", subcore_axis_name="s")


@functools.partial(
    pl.kernel,
    mesh=_mesh,
    out_type=jax.ShapeDtypeStruct((_ROWS, EMBED_DIM), jnp.float32),
    scratch_types=[
        pltpu.VMEM((_NCH, _CHUNK), jnp.int32),
        pltpu.VMEM((_NBUF, _CHUNK, EMBED_DIM), jnp.float32),
    ]
    + [pltpu.SemaphoreType.DMA] * (2 * _NBUF),
)
def _emb_lookup(idx_hbm, table_hbm, out_hbm, idx_v, rows_v, *sems):
    gsem = sems[:_NBUF]
    ssem = sems[_NBUF:]
    wid = lax.axis_index("s") * _NC + lax.axis_index("c")
    base = wid * _RPW
    pltpu.sync_copy(idx_hbm.at[wid], idx_v)

    def gather(c, s):
        pltpu.async_copy(table_hbm.at[idx_v.at[c]], rows_v.at[s], gsem[s])

    def gather_wait(s):
        pltpu.make_async_copy(table_hbm.at[idx_v.at[0]], rows_v.at[s],
                              gsem[s]).wait()

    def store(c, s):
        dst = out_hbm.at[pl.ds(base + c * _CHUNK, _CHUNK)]
        pltpu.async_copy(rows_v.at[s], dst, ssem[s])

    def store_wait(s):
        dst = out_hbm.at[pl.ds(base, _CHUNK)]
        pltpu.make_async_copy(rows_v.at[s], dst, ssem[s]).wait()

    for s in range(_NBUF):
        gather(s, s)

    def round_body(h, carry):
        c0 = _NBUF * h
        for s in range(_NBUF):
            gather_wait(s)
            store(c0 + s, s)
        for s in range(_NBUF):
            store_wait(s)
            gather(c0 + s + _NBUF, s)
        return carry

    lax.fori_loop(0, _NROUNDS - 1, round_body, 0)

    c0 = _NBUF * (_NROUNDS - 1)
    for s in range(_NBUF):
        gather_wait(s)
        store(c0 + s, s)
    for s in range(_NBUF):
        store_wait(s)


def kernel(indices, embedding_matrix):
    idx = indices.reshape(_NW, _NCH, _CHUNK).astype(jnp.int32)
    out = _emb_lookup(idx, embedding_matrix)
    return out.reshape(BATCH, HIST_LEN, EMBED_DIM)


# speedup vs baseline: 1.7747x; 1.7747x over previous
import functools

import jax
import jax.numpy as jnp
from jax import lax
from jax.experimental import pallas as pl
from jax.experimental.pallas import tpu as pltpu
from jax.experimental.pallas import tpu_sc as plsc

VOCAB = 100000
EMBED_DIM = 128
BATCH = 4096
HIST_LEN = 50

_NC = 2
_NS = 16
_NW = _NC * _NS
_BPW = BATCH // _NW
_BPC = 2
_CHUNK = _BPC * HIST_LEN
_NCH = _BPW // _BPC
_NBUF = 4
_NROUNDS = _NCH // _NBUF

_mesh = plsc.VectorSubcoreMesh(core_axis_name="c", subcore_axis_name="s")


@functools.partial(
    pl.kernel,
    mesh=_mesh,
    out_type=jax.ShapeDtypeStruct((BATCH, HIST_LEN, EMBED_DIM), jnp.float32),
    scratch_types=[
        pltpu.VMEM((_NCH, _CHUNK), jnp.int32),
        pltpu.VMEM((_NBUF, _CHUNK, EMBED_DIM), jnp.float32),
    ]
    + [pltpu.SemaphoreType.DMA] * (2 * _NBUF),
)
def _emb_lookup(idx_hbm, table_hbm, out_hbm, idx_v, rows_v, *sems):
    gsem = sems[:_NBUF]
    ssem = sems[_NBUF:]
    wid = lax.axis_index("s") * _NC + lax.axis_index("c")
    bbase = wid * _BPW
    pltpu.sync_copy(idx_hbm.at[wid], idx_v)

    def gather(c, s):
        pltpu.async_copy(table_hbm.at[idx_v.at[c]], rows_v.at[s], gsem[s])

    def gather_wait(s):
        pltpu.make_async_copy(table_hbm.at[idx_v.at[0]], rows_v.at[s],
                              gsem[s]).wait()

    def store(c, s):
        b = bbase + c * _BPC
        buf = rows_v.at[s]
        pltpu.async_copy(buf.at[pl.ds(0, HIST_LEN)], out_hbm.at[b], ssem[s])
        pltpu.async_copy(buf.at[pl.ds(HIST_LEN, HIST_LEN)], out_hbm.at[b + 1],
                         ssem[s])

    def store_wait(s):
        buf = rows_v.at[s]
        pltpu.make_async_copy(buf.at[pl.ds(0, HIST_LEN)], out_hbm.at[0],
                              ssem[s]).wait()
        pltpu.make_async_copy(buf.at[pl.ds(0, HIST_LEN)], out_hbm.at[0],
                              ssem[s]).wait()

    for s in range(_NBUF):
        gather(s, s)

    def round_body(h, carry):
        c0 = _NBUF * h
        for s in range(_NBUF):
            gather_wait(s)
            store(c0 + s, s)
        for s in range(_NBUF):
            store_wait(s)
            gather(c0 + s + _NBUF, s)
        return carry

    lax.fori_loop(0, _NROUNDS - 1, round_body, 0)

    c0 = _NBUF * (_NROUNDS - 1)
    for s in range(_NBUF):
        gather_wait(s)
        store(c0 + s, s)
    for s in range(_NBUF):
        store_wait(s)


def kernel(indices, embedding_matrix):
    idx = indices.reshape(_NW, _NCH, _CHUNK).astype(jnp.int32)
    return _emb_lookup(idx, embedding_matrix)


# speedup vs baseline: 3.0584x; 1.7233x over previous
import functools

import jax
import jax.numpy as jnp
from jax import lax
from jax.experimental import pallas as pl
from jax.experimental.pallas import tpu as pltpu
from jax.experimental.pallas import tpu_sc as plsc

VOCAB = 100000
EMBED_DIM = 128
BATCH = 4096
HIST_LEN = 50

_NC = 2
_NS = 16
_NW = _NC * _NS
_ROWS = BATCH * HIST_LEN
_RPW = _ROWS // _NW
_CHUNK = 128
_NCH = _RPW // _CHUNK
_NBUF = 5
_NROUNDS = _NCH // _NBUF

_mesh = plsc.VectorSubcoreMesh(core_axis_name="c", subcore_axis_name="s")


@functools.partial(
    pl.kernel,
    mesh=_mesh,
    out_type=jax.ShapeDtypeStruct((_ROWS, EMBED_DIM), jnp.float32),
    scratch_types=[
        pltpu.VMEM((_NCH, _CHUNK), jnp.int32),
        pltpu.VMEM((_NBUF, _CHUNK, EMBED_DIM), jnp.float32),
    ]
    + [pltpu.SemaphoreType.DMA] * (2 * _NBUF),
)
def _emb_lookup(idx_hbm, table_hbm, out_hbm, idx_v, rows_v, *sems):
    gsem = sems[:_NBUF]
    ssem = sems[_NBUF:]
    wid = lax.axis_index("s") * _NC + lax.axis_index("c")
    base = wid * _RPW
    pltpu.sync_copy(idx_hbm.at[wid], idx_v)

    def gather(c, s):
        pltpu.async_copy(table_hbm.at[idx_v.at[c]], rows_v.at[s], gsem[s])

    def gather_wait(s):
        pltpu.make_async_copy(table_hbm.at[idx_v.at[0]], rows_v.at[s],
                              gsem[s]).wait()

    def store(c, s):
        dst = out_hbm.at[pl.ds(base + c * _CHUNK, _CHUNK)]
        pltpu.async_copy(rows_v.at[s], dst, ssem[s])

    def store_wait(s):
        dst = out_hbm.at[pl.ds(base, _CHUNK)]
        pltpu.make_async_copy(rows_v.at[s], dst, ssem[s]).wait()

    for s in range(_NBUF):
        gather(s, s)

    def round_body(h, carry):
        c0 = _NBUF * h
        for s in range(_NBUF):
            gather_wait(s)
            store(c0 + s, s)
        for s in range(_NBUF):
            store_wait(s)
            gather(c0 + s + _NBUF, s)
        return carry

    lax.fori_loop(0, _NROUNDS - 1, round_body, 0)

    c0 = _NBUF * (_NROUNDS - 1)
    for s in range(_NBUF):
        gather_wait(s)
        store(c0 + s, s)
    for s in range(_NBUF):
        store_wait(s)


def kernel(indices, embedding_matrix):
    idx = indices.T.reshape(_NW, _NCH, _CHUNK).astype(jnp.int32)
    out = _emb_lookup(idx, embedding_matrix)
    return out.reshape(HIST_LEN, BATCH, EMBED_DIM).transpose(1, 0, 2)


# speedup vs baseline: 3.1135x; 1.0180x over previous
import functools

import jax
import jax.numpy as jnp
from jax import lax
from jax.experimental import pallas as pl
from jax.experimental.pallas import tpu as pltpu
from jax.experimental.pallas import tpu_sc as plsc

VOCAB = 100000
EMBED_DIM = 128
BATCH = 4096
HIST_LEN = 50

_NC = 2
_NS = 16
_NW = _NC * _NS
_ROWS = BATCH * HIST_LEN
_CHUNK = 128
_NCH = HIST_LEN
_NBUF = 5
_NROUNDS = _NCH // _NBUF

_mesh = plsc.VectorSubcoreMesh(core_axis_name="c", subcore_axis_name="s")


@functools.partial(
    pl.kernel,
    mesh=_mesh,
    out_type=jax.ShapeDtypeStruct((_ROWS, EMBED_DIM), jnp.float32),
    scratch_types=[
        pltpu.VMEM((_NCH, _CHUNK), jnp.int32),
        pltpu.VMEM((_NBUF, _CHUNK, EMBED_DIM), jnp.float32),
    ]
    + [pltpu.SemaphoreType.DMA] * (2 * _NBUF),
)
def _emb_lookup(idx_hbm, table_hbm, out_hbm, idx_v, rows_v, *sems):
    gsem = sems[:_NBUF]
    ssem = sems[_NBUF:]
    wid = lax.axis_index("s") * _NC + lax.axis_index("c")
    bbase = wid * _CHUNK
    pltpu.sync_copy(idx_hbm.at[:, pl.ds(bbase, _CHUNK)], idx_v)

    def gather(c, s):
        pltpu.async_copy(table_hbm.at[idx_v.at[c]], rows_v.at[s], gsem[s])

    def gather_wait(s):
        pltpu.make_async_copy(table_hbm.at[idx_v.at[0]], rows_v.at[s],
                              gsem[s]).wait()

    def store(c, s):
        dst = out_hbm.at[pl.ds(c * BATCH + bbase, _CHUNK)]
        pltpu.async_copy(rows_v.at[s], dst, ssem[s])

    def store_wait(s):
        dst = out_hbm.at[pl.ds(bbase, _CHUNK)]
        pltpu.make_async_copy(rows_v.at[s], dst, ssem[s]).wait()

    for s in range(_NBUF):
        gather(s, s)

    def round_body(h, carry):
        c0 = _NBUF * h
        for s in range(_NBUF):
            gather_wait(s)
            store(c0 + s, s)
        for s in range(_NBUF):
            store_wait(s)
            gather(c0 + s + _NBUF, s)
        return carry

    lax.fori_loop(0, _NROUNDS - 1, round_body, 0)

    c0 = _NBUF * (_NROUNDS - 1)
    for s in range(_NBUF):
        gather_wait(s)
        store(c0 + s, s)
    for s in range(_NBUF):
        store_wait(s)


def kernel(indices, embedding_matrix):
    out = _emb_lookup(indices.T.astype(jnp.int32), embedding_matrix)
    return out.reshape(HIST_LEN, BATCH, EMBED_DIM).transpose(1, 0, 2)


# speedup vs baseline: 3.2216x; 1.0347x over previous
import functools

import jax
import jax.numpy as jnp
from jax import lax
from jax.experimental import pallas as pl
from jax.experimental.pallas import tpu as pltpu
from jax.experimental.pallas import tpu_sc as plsc

VOCAB = 100000
EMBED_DIM = 128
BATCH = 4096
HIST_LEN = 50

_NC = 2
_NS = 16
_NW = _NC * _NS
_ROWS = BATCH * HIST_LEN
_COLS = 128
_CHUNK = 64
_NCH = HIST_LEN * (_COLS // _CHUNK)
_NBUF = 10
_NROUNDS = _NCH // _NBUF

_mesh = plsc.VectorSubcoreMesh(core_axis_name="c", subcore_axis_name="s")


@functools.partial(
    pl.kernel,
    mesh=_mesh,
    out_type=jax.ShapeDtypeStruct((_ROWS, EMBED_DIM), jnp.float32),
    scratch_types=[
        pltpu.VMEM((HIST_LEN, _COLS), jnp.int32),
        pltpu.VMEM((_NBUF, _CHUNK, EMBED_DIM), jnp.float32),
    ]
    + [pltpu.SemaphoreType.DMA] * (2 * _NBUF),
)
def _emb_lookup(idx_hbm, table_hbm, out_hbm, idx_v, rows_v, *sems):
    gsem = sems[:_NBUF]
    ssem = sems[_NBUF:]
    wid = lax.axis_index("s") * _NC + lax.axis_index("c")
    bbase = wid * _COLS
    pltpu.sync_copy(idx_hbm.at[:, pl.ds(bbase, _COLS)], idx_v)

    def gather(t, half, s):
        idx = idx_v.at[t, pl.ds(half * _CHUNK, _CHUNK)]
        pltpu.async_copy(table_hbm.at[idx], rows_v.at[s], gsem[s])

    def gather_wait(s):
        idx = idx_v.at[0, pl.ds(0, _CHUNK)]
        pltpu.make_async_copy(table_hbm.at[idx], rows_v.at[s], gsem[s]).wait()

    def store(t, half, s):
        dst = out_hbm.at[pl.ds(t * BATCH + bbase + half * _CHUNK, _CHUNK)]
        pltpu.async_copy(rows_v.at[s], dst, ssem[s])

    def store_wait(s):
        dst = out_hbm.at[pl.ds(bbase, _CHUNK)]
        pltpu.make_async_copy(rows_v.at[s], dst, ssem[s]).wait()

    for s in range(_NBUF):
        gather(s // 2, s % 2, s)

    def round_body(h, carry):
        c0 = _NBUF * h
        for s in range(_NBUF):
            gather_wait(s)
            store((c0 + s) // 2, s % 2, s)
        for s in range(_NBUF):
            store_wait(s)
            gather((c0 + s + _NBUF) // 2, s % 2, s)
        return carry

    lax.fori_loop(0, _NROUNDS - 1, round_body, 0)

    c0 = _NBUF * (_NROUNDS - 1)
    for s in range(_NBUF):
        gather_wait(s)
        store((c0 + s) // 2, s % 2, s)
    for s in range(_NBUF):
        store_wait(s)


def kernel(indices, embedding_matrix):
    out = _emb_lookup(indices.T.astype(jnp.int32), embedding_matrix)
    return out.reshape(HIST_LEN, BATCH, EMBED_DIM).transpose(1, 0, 2)
